# baseline (speedup 1.0000x reference)
import functools

import jax
import jax.numpy as jnp
from jax.experimental import pallas as pl

N = 10000
D_IN = 128
D_OUT = 128
BM = 400


def _support_body(x_ref, w_ref, o_ref):
    o_ref[...] = jnp.dot(x_ref[...], w_ref[...],
                         preferred_element_type=jnp.float32)


def _spmm_body(adj_ref, s_ref, b_ref, o_ref):
    acc = jnp.dot(adj_ref[...], s_ref[...],
                  preferred_element_type=jnp.float32)
    o_ref[...] = acc + b_ref[...]


@functools.partial(jax.jit, static_argnames=())
def kernel(input, adj, W, b):
    support = pl.pallas_call(
        _support_body,
        out_shape=jax.ShapeDtypeStruct((N, D_OUT), jnp.float32),
    )(input, W)

    b2 = b.reshape(1, D_OUT)
    grid = (N // BM,)
    out = pl.pallas_call(
        _spmm_body,
        grid=grid,
        in_specs=[
            pl.BlockSpec((BM, N), lambda i: (i, 0)),
            pl.BlockSpec((N, D_OUT), lambda i: (0, 0)),
            pl.BlockSpec((1, D_OUT), lambda i: (0, 0)),
        ],
        out_specs=pl.BlockSpec((BM, D_OUT), lambda i: (i, 0)),
        out_shape=jax.ShapeDtypeStruct((N, D_OUT), jnp.float32),
    )(adj, support, b2)
    return out


# speedup vs baseline: 1.0454x; 1.0454x over previous
import functools

import jax
import jax.numpy as jnp
from jax.experimental import pallas as pl
from jax.experimental.pallas import tpu as pltpu

N = 10000
D_IN = 128
D_OUT = 128
BM = 400


def _gcn_body(x_ref, w_ref, b_ref, adj_ref, o_ref, s_ref):
    @pl.when(pl.program_id(0) == 0)
    def _():
        s_ref[...] = jnp.dot(x_ref[...], w_ref[...],
                             preferred_element_type=jnp.float32)

    acc = jnp.dot(adj_ref[...], s_ref[...],
                  preferred_element_type=jnp.float32)
    o_ref[...] = acc + b_ref[...]


@functools.partial(jax.jit, static_argnames=())
def kernel(input, adj, W, b):
    b2 = b.reshape(1, D_OUT)
    grid = (N // BM,)
    out = pl.pallas_call(
        _gcn_body,
        grid=grid,
        in_specs=[
            pl.BlockSpec((N, D_IN), lambda i: (0, 0)),
            pl.BlockSpec((D_IN, D_OUT), lambda i: (0, 0)),
            pl.BlockSpec((1, D_OUT), lambda i: (0, 0)),
            pl.BlockSpec((BM, N), lambda i: (i, 0)),
        ],
        out_specs=pl.BlockSpec((BM, D_OUT), lambda i: (i, 0)),
        out_shape=jax.ShapeDtypeStruct((N, D_OUT), jnp.float32),
        scratch_shapes=[pltpu.VMEM((N, D_OUT), jnp.float32)],
        compiler_params=pltpu.CompilerParams(
            dimension_semantics=("arbitrary",),
        ),
    )(input, W, b2, adj)
    return out
